# Initial kernel scaffold; baseline (speedup 1.0000x reference)
import jax
import jax.numpy as jnp
from jax.experimental import pallas as pl


def kernel(corr_index, nei_index, lstm_state, wse, bse, w1, b1, w2, b2):
    raise NotImplementedError("write your pallas kernel here")


# speedup vs baseline: 5.4538x; 5.4538x over previous
import jax
import jax.numpy as jnp
import numpy as np
from jax.experimental import pallas as pl
from jax.experimental.pallas import tpu as pltpu

_R = 64
_P = 8
_SB = 4
_BIG = 1e9


def _body(SB, N, H):
    J8 = N // _P
    M = SB * N * J8

    def body(corr_ref, lstm8_ref, lstm_ref, nei_ref, wc_ref, whj_ref,
             whi_ref, w2_ref, emb_ref, fs_ref, out_ref):
        f32 = jnp.float32
        c = jnp.dot(corr_ref[...].reshape(M, 2 * _P), wc_ref[...],
                    preferred_element_type=f32)
        ap = jnp.dot(lstm8_ref[...].reshape(SB * J8, _P * H), whj_ref[...],
                     preferred_element_type=f32)
        ai = jnp.dot(lstm_ref[...].reshape(SB * N, H), whi_ref[...],
                     preferred_element_type=f32) + fs_ref[0:1, :]

        pre = (c.reshape(SB, N, J8, _P * _R)
               + ap.reshape(SB, 1, J8, _P * _R)
               + ai.reshape(SB, N, 1, _P * _R))
        x = jnp.maximum(pre, 0.0).astype(jnp.bfloat16).reshape(M, _P * _R)

        y = jnp.dot(x, w2_ref[...], preferred_element_type=f32)
        nm = (nei_ref[...].reshape(M, _P) <= 0.0).astype(jnp.bfloat16)
        mt = jnp.dot(nm, emb_ref[...], preferred_element_type=f32)

        z = (y + mt).reshape(SB, N, J8, _P * 32)
        pm = jnp.max(z, axis=2)
        pm = jnp.maximum(pm[:, :, :128], pm[:, :, 128:])
        pm = jnp.maximum(pm[:, :, :64], pm[:, :, 64:])
        pm = jnp.maximum(pm[:, :, :32], pm[:, :, 32:])
        out_ref[...] = jnp.maximum(pm + fs_ref[1:2, :H].reshape(1, 1, H), 0.0)

    return body


def _pack(wse, bse, w1, b1, w2, E, H):
    bf16 = jnp.bfloat16
    wc = wse @ w1[:E]
    b1f = bse @ w1[:E] + b1
    whj = w1[E:E + H]
    whi = w1[E + H:]

    wc8 = jnp.zeros((2 * _P, _P * _R), jnp.float32)
    whj8 = jnp.zeros((_P * H, _P * _R), jnp.float32)
    w2o = jnp.zeros((_P * _R, _P * 32), jnp.float32)
    emb = jnp.zeros((_P, _P * 32), jnp.float32)
    for k in range(_P):
        wc8 = wc8.at[2 * k, _R * k:_R * (k + 1)].set(wc[0])
        wc8 = wc8.at[2 * k + 1, _R * k:_R * (k + 1)].set(wc[1])
        whj8 = whj8.at[H * k:H * (k + 1), _R * k:_R * (k + 1)].set(whj)
        w2o = w2o.at[_R * k:_R * (k + 1), 32 * k:32 * (k + 1)].set(w2)
        emb = emb.at[k, 32 * k:32 * (k + 1)].set(-_BIG)
    whid = jnp.tile(whi, (1, _P))

    fs = jnp.zeros((8, _P * _R), jnp.float32)
    fs = fs.at[0:1, :].set(jnp.tile(b1f, (1, _P)))
    return (wc8.astype(bf16), whj8.astype(bf16), whid.astype(bf16),
            w2o.astype(bf16), emb.astype(bf16), fs)


def kernel(corr_index, nei_index, lstm_state, wse, bse, w1, b1, w2, b2):
    single = corr_index.ndim == 3
    if single:
        corr_index, nei_index, lstm_state = (
            corr_index[None], nei_index[None], lstm_state[None])
    B, N = corr_index.shape[0], corr_index.shape[1]
    H = lstm_state.shape[-1]
    E = wse.shape[1]
    assert N % _P == 0 and B % _SB == 0
    J8 = N // _P
    bf16 = jnp.bfloat16

    wc8, whj8, whid, w2o, emb, fs = _pack(wse, bse, w1, b1, w2, E, H)
    fs = fs.at[1:2, :H].set(b2.reshape(1, H))

    corr8 = corr_index.reshape(B, N * J8, 2 * _P).astype(bf16)
    lstm8 = lstm_state.reshape(B, J8, _P * H).astype(bf16)
    lstmb = lstm_state.astype(bf16)
    nei8 = nei_index.reshape(B, N * J8, _P)

    grid_spec = pltpu.PrefetchScalarGridSpec(
        num_scalar_prefetch=0,
        grid=(B // _SB,),
        in_specs=[
            pl.BlockSpec((_SB, N * J8, 2 * _P), lambda b: (b, 0, 0)),
            pl.BlockSpec((_SB, J8, _P * H), lambda b: (b, 0, 0)),
            pl.BlockSpec((_SB, N, H), lambda b: (b, 0, 0)),
            pl.BlockSpec((_SB, N * J8, _P), lambda b: (b, 0, 0)),
            pl.BlockSpec(wc8.shape, lambda b: (0, 0)),
            pl.BlockSpec(whj8.shape, lambda b: (0, 0)),
            pl.BlockSpec(whid.shape, lambda b: (0, 0)),
            pl.BlockSpec(w2o.shape, lambda b: (0, 0)),
            pl.BlockSpec(emb.shape, lambda b: (0, 0)),
            pl.BlockSpec(fs.shape, lambda b: (0, 0)),
        ],
        out_specs=pl.BlockSpec((_SB, N, H), lambda b: (b, 0, 0)),
    )

    out = pl.pallas_call(
        _body(_SB, N, H),
        out_shape=jax.ShapeDtypeStruct((B, N, H), jnp.float32),
        grid_spec=grid_spec,
        compiler_params=pltpu.CompilerParams(
            dimension_semantics=("parallel",)),
    )(corr8, lstm8, lstmb, nei8, wc8, whj8, whid, w2o, emb, fs)

    return out[0] if single else out


# speedup vs baseline: 5.6533x; 1.0366x over previous
import jax
import jax.numpy as jnp
import numpy as np
from jax.experimental import pallas as pl
from jax.experimental.pallas import tpu as pltpu

_R = 64
_P = 8
_SB = 8
_BIG = 1e9


def _body(SB, N, H):
    J8 = N // _P
    M = SB * N * J8

    def body(corr_ref, lstm8_ref, lstm_ref, nei_ref, wc_ref, whj_ref,
             whi_ref, w2_ref, emb_ref, fs_ref, out_ref):
        f32 = jnp.float32
        c = jnp.dot(corr_ref[...].reshape(M, 2 * _P), wc_ref[...],
                    preferred_element_type=f32)
        ap = jnp.dot(lstm8_ref[...].reshape(SB * J8, _P * H), whj_ref[...],
                     preferred_element_type=f32)
        ai = jnp.dot(lstm_ref[...].reshape(SB * N, H), whi_ref[...],
                     preferred_element_type=f32) + fs_ref[0:1, :]

        pre = (c.reshape(SB, N, J8, _P * _R)
               + ap.reshape(SB, 1, J8, _P * _R)
               + ai.reshape(SB, N, 1, _P * _R))
        x = jnp.maximum(pre, 0.0).astype(jnp.bfloat16).reshape(M, _P * _R)

        y = jnp.dot(x, w2_ref[...], preferred_element_type=f32)
        nm = (nei_ref[...].reshape(M, _P) <= 0.0).astype(jnp.bfloat16)
        mt = jnp.dot(nm, emb_ref[...], preferred_element_type=f32)

        z = (y + mt).reshape(SB, N, J8, _P * 32)
        pm = jnp.max(z, axis=2)
        pm = jnp.maximum(pm[:, :, :128], pm[:, :, 128:])
        pm = jnp.maximum(pm[:, :, :64], pm[:, :, 64:])
        pm = jnp.maximum(pm[:, :, :32], pm[:, :, 32:])
        out_ref[...] = jnp.maximum(pm + fs_ref[1:2, :H].reshape(1, 1, H), 0.0)

    return body


def _pack(wse, bse, w1, b1, w2, E, H):
    bf16 = jnp.bfloat16
    wc = wse @ w1[:E]
    b1f = bse @ w1[:E] + b1
    whj = w1[E:E + H]
    whi = w1[E + H:]

    wc8 = jnp.zeros((2 * _P, _P * _R), jnp.float32)
    whj8 = jnp.zeros((_P * H, _P * _R), jnp.float32)
    w2o = jnp.zeros((_P * _R, _P * 32), jnp.float32)
    emb = jnp.zeros((_P, _P * 32), jnp.float32)
    for k in range(_P):
        wc8 = wc8.at[2 * k, _R * k:_R * (k + 1)].set(wc[0])
        wc8 = wc8.at[2 * k + 1, _R * k:_R * (k + 1)].set(wc[1])
        whj8 = whj8.at[H * k:H * (k + 1), _R * k:_R * (k + 1)].set(whj)
        w2o = w2o.at[_R * k:_R * (k + 1), 32 * k:32 * (k + 1)].set(w2)
        emb = emb.at[k, 32 * k:32 * (k + 1)].set(-_BIG)
    whid = jnp.tile(whi, (1, _P))

    fs = jnp.zeros((8, _P * _R), jnp.float32)
    fs = fs.at[0:1, :].set(jnp.tile(b1f, (1, _P)))
    return (wc8.astype(bf16), whj8.astype(bf16), whid.astype(bf16),
            w2o.astype(bf16), emb.astype(bf16), fs)


def kernel(corr_index, nei_index, lstm_state, wse, bse, w1, b1, w2, b2):
    single = corr_index.ndim == 3
    if single:
        corr_index, nei_index, lstm_state = (
            corr_index[None], nei_index[None], lstm_state[None])
    B, N = corr_index.shape[0], corr_index.shape[1]
    H = lstm_state.shape[-1]
    E = wse.shape[1]
    assert N % _P == 0 and B % _SB == 0
    J8 = N // _P
    bf16 = jnp.bfloat16

    wc8, whj8, whid, w2o, emb, fs = _pack(wse, bse, w1, b1, w2, E, H)
    fs = fs.at[1:2, :H].set(b2.reshape(1, H))

    corr8 = corr_index.reshape(B, N * J8, 2 * _P).astype(bf16)
    lstm8 = lstm_state.reshape(B, J8, _P * H).astype(bf16)
    lstmb = lstm_state.astype(bf16)
    nei8 = nei_index.reshape(B, N * J8, _P)

    grid_spec = pltpu.PrefetchScalarGridSpec(
        num_scalar_prefetch=0,
        grid=(B // _SB,),
        in_specs=[
            pl.BlockSpec((_SB, N * J8, 2 * _P), lambda b: (b, 0, 0)),
            pl.BlockSpec((_SB, J8, _P * H), lambda b: (b, 0, 0)),
            pl.BlockSpec((_SB, N, H), lambda b: (b, 0, 0)),
            pl.BlockSpec((_SB, N * J8, _P), lambda b: (b, 0, 0)),
            pl.BlockSpec(wc8.shape, lambda b: (0, 0)),
            pl.BlockSpec(whj8.shape, lambda b: (0, 0)),
            pl.BlockSpec(whid.shape, lambda b: (0, 0)),
            pl.BlockSpec(w2o.shape, lambda b: (0, 0)),
            pl.BlockSpec(emb.shape, lambda b: (0, 0)),
            pl.BlockSpec(fs.shape, lambda b: (0, 0)),
        ],
        out_specs=pl.BlockSpec((_SB, N, H), lambda b: (b, 0, 0)),
    )

    out = pl.pallas_call(
        _body(_SB, N, H),
        out_shape=jax.ShapeDtypeStruct((B, N, H), jnp.float32),
        grid_spec=grid_spec,
        compiler_params=pltpu.CompilerParams(
            dimension_semantics=("arbitrary",)),
    )(corr8, lstm8, lstmb, nei8, wc8, whj8, whid, w2o, emb, fs)

    return out[0] if single else out
